# baseline (speedup 1.0000x reference)
import functools
import math

import jax
import jax.numpy as jnp
from jax import lax
from jax.experimental import pallas as pl
from jax.experimental.pallas import tpu as pltpu
from jax.experimental.pallas import tpu_sc as plsc

VOCAB = 1000000
D = 64
ROWS = 4096
COLS = 50
B = ROWS * COLS
NC = 2
NS = 16
NW = NC * NS
B_PER_W = B // NW
CHUNK = 800
NCHUNK = B_PER_W // CHUNK
SCALE = math.sqrt(D)


def _body(x_hbm, emb_hbm, out_hbm, idx_v, rows_v, sem):
    wid = lax.axis_index("s") * NC + lax.axis_index("c")
    base = wid * B_PER_W

    def step(c, carry):
        off = base + c * CHUNK
        pltpu.sync_copy(x_hbm.at[pl.ds(off, CHUNK)], idx_v)
        pltpu.async_copy(emb_hbm.at[idx_v], rows_v, sem).wait()

        def scale_row(r, carry2):
            for j in range(D // 16):
                sl = pl.ds(j * 16, 16)
                rows_v[r, sl] = rows_v[r, sl] * SCALE
            return carry2

        lax.fori_loop(0, CHUNK, scale_row, 0, unroll=4)
        pltpu.sync_copy(rows_v, out_hbm.at[pl.ds(off, CHUNK)])
        return carry

    lax.fori_loop(0, NCHUNK, step, 0)


@jax.jit
def kernel(x, embedding):
    xf = x.reshape(B).astype(jnp.int32)
    run = pl.kernel(
        _body,
        out_type=jax.ShapeDtypeStruct((B, D), jnp.float32),
        mesh=plsc.VectorSubcoreMesh(core_axis_name="c", subcore_axis_name="s"),
        scratch_types=[
            pltpu.VMEM((CHUNK,), jnp.int32),
            pltpu.VMEM((CHUNK, D), jnp.float32),
            pltpu.SemaphoreType.DMA,
        ],
        compiler_params=pltpu.CompilerParams(use_tc_tiling_on_sc=False),
    )
    out = run(xf, embedding)
    return out.reshape(ROWS, COLS, D)


# speedup vs baseline: 1.0239x; 1.0239x over previous
import math

import jax
import jax.numpy as jnp
from jax import lax
from jax.experimental import pallas as pl
from jax.experimental.pallas import tpu as pltpu
from jax.experimental.pallas import tpu_sc as plsc

VOCAB = 1000000
D = 64
ROWS = 4096
COLS = 50
B = ROWS * COLS
NC = 2
NS = 16
NW = NC * NS
B_PER_W = B // NW
CHUNK = 800
NCHUNK = B_PER_W // CHUNK
SCALE = math.sqrt(D)


def _body(x_hbm, emb_hbm, out_hbm, idx_v, rows0, rows1, gsem, osem):
    wid = lax.axis_index("s") * NC + lax.axis_index("c")
    base = wid * B_PER_W
    bufs = (rows0, rows1)

    pltpu.sync_copy(x_hbm.at[pl.ds(base, B_PER_W)], idx_v)

    def gather(c, buf):
        return pltpu.async_copy(
            emb_hbm.at[idx_v.at[pl.ds(c * CHUNK, CHUNK)]], buf, gsem)

    def wait_gather(c, buf):
        pltpu.make_async_copy(
            emb_hbm.at[idx_v.at[pl.ds(c * CHUNK, CHUNK)]], buf, gsem).wait()

    def put(c, buf):
        return pltpu.async_copy(
            buf, out_hbm.at[pl.ds(base + c * CHUNK, CHUNK)], osem)

    def wait_put(c, buf):
        pltpu.make_async_copy(
            buf, out_hbm.at[pl.ds(base + c * CHUNK, CHUNK)], osem).wait()

    def scale(buf):
        def scale_row(r, carry):
            for j in range(D // 16):
                sl = pl.ds(j * 16, 16)
                buf[r, sl] = buf[r, sl] * SCALE
            return carry
        lax.fori_loop(0, CHUNK, scale_row, 0, unroll=8)

    gather(0, bufs[0])
    for c in range(NCHUNK):
        cur = bufs[c % 2]
        nxt = bufs[(c + 1) % 2]
        if c + 1 < NCHUNK:
            if c >= 1:
                wait_put(c - 1, nxt)
            gather(c + 1, nxt)
        wait_gather(c, cur)
        scale(cur)
        put(c, cur)
    wait_put(NCHUNK - 2, bufs[(NCHUNK - 2) % 2])
    wait_put(NCHUNK - 1, bufs[(NCHUNK - 1) % 2])


@jax.jit
def kernel(x, embedding):
    xf = x.reshape(B).astype(jnp.int32)
    run = pl.kernel(
        _body,
        out_type=jax.ShapeDtypeStruct((B, D), jnp.float32),
        mesh=plsc.VectorSubcoreMesh(core_axis_name="c", subcore_axis_name="s"),
        scratch_types=[
            pltpu.VMEM((B_PER_W,), jnp.int32),
            pltpu.VMEM((CHUNK, D), jnp.float32),
            pltpu.VMEM((CHUNK, D), jnp.float32),
            pltpu.SemaphoreType.DMA,
            pltpu.SemaphoreType.DMA,
        ],
        compiler_params=pltpu.CompilerParams(use_tc_tiling_on_sc=False),
    )
    out = run(xf, embedding)
    return out.reshape(ROWS, COLS, D)
